# baseline (speedup 1.0000x reference)
import functools

import jax
import jax.numpy as jnp
from jax import lax
from jax.experimental import pallas as pl
from jax.experimental.pallas import tpu as pltpu
from jax.experimental.pallas import tpu_sc as plsc

L_BITS = 20
BATCH = 16384
NUM_CORES = 2
NUM_SUBCORES = 16
NUM_WORKERS = NUM_CORES * NUM_SUBCORES
B_W = BATCH // NUM_WORKERS
CHUNK = 128
N_CHUNKS = B_W // CHUNK
LANES = 16


def _ewf_body(x_hbm, aux_hbm, out_hbm, x_v, idx_v, out_v, sem):
    cid = lax.axis_index("c")
    sid = lax.axis_index("s")
    wid = sid * NUM_CORES + cid
    base = wid * B_W

    pltpu.sync_copy(x_hbm.at[pl.ds(base * L_BITS, B_W * L_BITS)], x_v)

    lanes = lax.iota(jnp.int32, LANES)

    def group(g, carry):
        rows = g * LANES + lanes
        flat = rows * L_BITS
        acc = jnp.zeros((LANES,), jnp.int32)
        for j in range(L_BITS):
            vals = plsc.load_gather(x_v, [flat + j])
            w = jnp.full((LANES,), 1 << (L_BITS - 1 - j), dtype=jnp.int32)
            acc = acc + jnp.where(vals > 0.0, w, jnp.zeros((LANES,), jnp.int32))
        plsc.store_scatter(idx_v, [rows // CHUNK, rows % CHUNK], acc)
        return carry

    lax.fori_loop(0, B_W // LANES, group, 0)

    copies = []
    for c in range(N_CHUNKS):
        copies.append(
            pltpu.async_copy(
                aux_hbm.at[idx_v.at[c]],
                out_v.at[pl.ds(c * CHUNK, CHUNK)],
                sem,
            )
        )
    for cp in copies:
        cp.wait()

    pltpu.sync_copy(out_v, out_hbm.at[pl.ds(base, B_W)])


@jax.jit
def kernel(x, aux):
    mesh = plsc.VectorSubcoreMesh(core_axis_name="c", subcore_axis_name="s")
    run = pl.kernel(
        _ewf_body,
        out_type=jax.ShapeDtypeStruct((BATCH,), jnp.float32),
        mesh=mesh,
        compiler_params=pltpu.CompilerParams(needs_layout_passes=False),
        scratch_types=[
            pltpu.VMEM((B_W * L_BITS,), jnp.float32),
            pltpu.VMEM((N_CHUNKS, CHUNK), jnp.int32),
            pltpu.VMEM((B_W,), jnp.float32),
            pltpu.SemaphoreType.DMA,
        ],
    )
    return run(x.reshape(-1), aux)


# speedup vs baseline: 1.1265x; 1.1265x over previous
import functools

import jax
import jax.numpy as jnp
from jax import lax
from jax.experimental import pallas as pl
from jax.experimental.pallas import tpu as pltpu
from jax.experimental.pallas import tpu_sc as plsc

L_BITS = 20
BATCH = 16384
NUM_CORES = 2
NUM_SUBCORES = 16
NUM_WORKERS = NUM_CORES * NUM_SUBCORES
B_W = BATCH // NUM_WORKERS
CHUNK = 128
N_CHUNKS = B_W // CHUNK
LANES = 16


def _ewf_body(x_hbm, aux_hbm, out_hbm, x_v, idx_v, out_v, sem):
    cid = lax.axis_index("c")
    sid = lax.axis_index("s")
    wid = sid * NUM_CORES + cid
    base = wid * B_W

    pltpu.sync_copy(x_hbm.at[pl.ds(base, B_W), :], x_v)

    lanes = lax.iota(jnp.int32, LANES)

    def group(g, carry):
        rows = g * LANES + lanes
        acc = jnp.zeros((LANES,), jnp.int32)
        for j in range(L_BITS):
            cols = jnp.full((LANES,), j, dtype=jnp.int32)
            vals = plsc.load_gather(x_v, [rows, cols])
            w = jnp.full((LANES,), 1 << (L_BITS - 1 - j), dtype=jnp.int32)
            acc = acc + jnp.where(vals > 0.0, w, jnp.zeros((LANES,), jnp.int32))
        plsc.store_scatter(idx_v, [rows // CHUNK, rows % CHUNK], acc)
        return carry

    lax.fori_loop(0, B_W // LANES, group, 0)

    copies = []
    for c in range(N_CHUNKS):
        copies.append(
            pltpu.async_copy(
                aux_hbm.at[idx_v.at[c]],
                out_v.at[pl.ds(c * CHUNK, CHUNK)],
                sem,
            )
        )
    for cp in copies:
        cp.wait()

    pltpu.sync_copy(out_v, out_hbm.at[pl.ds(base, B_W)])


@jax.jit
def kernel(x, aux):
    mesh = plsc.VectorSubcoreMesh(core_axis_name="c", subcore_axis_name="s")
    run = pl.kernel(
        _ewf_body,
        out_type=jax.ShapeDtypeStruct((BATCH,), jnp.float32),
        mesh=mesh,
        compiler_params=pltpu.CompilerParams(needs_layout_passes=False),
        scratch_types=[
            pltpu.VMEM((B_W, L_BITS), jnp.float32),
            pltpu.VMEM((N_CHUNKS, CHUNK), jnp.int32),
            pltpu.VMEM((B_W,), jnp.float32),
            pltpu.SemaphoreType.DMA,
        ],
    )
    return run(x, aux)
